# baseline (device time: 503762 ns/iter reference)
import jax
import jax.numpy as jnp
from jax import lax
from jax.experimental import pallas as pl
from jax.experimental.pallas import tpu as pltpu

M = 8192
D = 2048
EPS = 1e-6
Q = M // 4
NC = 8
R = Q // NC

YA = [0, 1, 2]
ZA = [3, 4, 5, 6, 7]
ZB = [2, 3, 4, 5, 6, 7]
XC = [0, 1]


def kernel(partial, resid, gamma):
    gamma2 = gamma.reshape(1, D)

    def body(p_ref, resid_ref, g_ref, o_ref,
             prec_buf, xrecv_buf, a_buf, p_st, r_st,
             ypart_s, yprec, x_s, xrecv,
             ya_s, ya_r, za_s, za_r, zb_s, zb_r, xc_s, xc_r,
             pin, rin, acp, bcp):
        x = lax.axis_index("x")
        y = lax.axis_index("y")
        z = lax.axis_index("z")
        xnbr = (1 - x, y, z)
        ynbr = (x, 1 - y, z)
        znbr = (x, y, 1 - z)
        h = jnp.bitwise_xor(y, z)
        a0 = (2 * x + h) * Q
        b0 = (2 * (1 - x) + h) * Q
        c0 = (2 * x + (1 - h)) * Q
        d0 = (2 * (1 - x) + (1 - h)) * Q

        def stage_in(c):
            slot = c % 2
            cp_p = pltpu.make_async_copy(
                p_ref.at[0, pl.ds(a0 + c * R, R), :],
                p_st.at[slot], pin.at[slot])
            cp_r = pltpu.make_async_copy(
                resid_ref.at[pl.ds(a0 + c * R, R), :],
                r_st.at[slot], rin.at[slot])
            cp_p.start()
            cp_r.start()
            return (cp_p, cp_r)

        pending = {0: stage_in(0)}

        bar = pltpu.get_barrier_semaphore()
        for nbr in (xnbr, ynbr, znbr):
            pl.semaphore_signal(bar, inc=1, device_id=nbr,
                                device_id_type=pl.DeviceIdType.MESH)
        pl.semaphore_wait(bar, 3)

        ypart_rdmas = []
        for c in range(NC):
            r = pltpu.make_async_remote_copy(
                src_ref=p_ref.at[0, pl.ds(c0 + c * R, R), :],
                dst_ref=prec_buf.at[c],
                send_sem=ypart_s.at[c],
                recv_sem=yprec.at[c],
                device_id=ynbr,
                device_id_type=pl.DeviceIdType.MESH,
            )
            r.start()
            ypart_rdmas.append(r)

        def xrecv_desc(b):
            return pltpu.make_async_remote_copy(
                src_ref=a_buf.at[b],
                dst_ref=xrecv_buf.at[b],
                send_sem=x_s.at[b],
                recv_sem=xrecv.at[b],
                device_id=xnbr,
                device_id_type=pl.DeviceIdType.MESH,
            )

        def forward_b(b):
            xrecv_desc(b).wait_recv()
            cp = pltpu.make_async_copy(
                xrecv_buf.at[b],
                o_ref.at[pl.ds(b0 + b * R, R), :],
                bcp.at[b])
            cp.start()
            zb = None
            if b in ZB:
                zb = pltpu.make_async_remote_copy(
                    src_ref=xrecv_buf.at[b],
                    dst_ref=o_ref.at[pl.ds(b0 + b * R, R), :],
                    send_sem=zb_s.at[ZB.index(b)],
                    recv_sem=zb_r.at[ZB.index(b)],
                    device_id=znbr,
                    device_id_type=pl.DeviceIdType.MESH,
                )
                zb.start()
            return (cp, zb)

        def ya_recv_desc(c):
            return pltpu.make_async_remote_copy(
                src_ref=a_buf.at[0],
                dst_ref=o_ref.at[pl.ds(c0 + c * R, R), :],
                send_sem=ypart_s.at[c],
                recv_sem=ya_r.at[YA.index(c)],
                device_id=ynbr,
                device_id_type=pl.DeviceIdType.MESH,
            )

        def forward_c(c):
            ya_recv_desc(c).wait_recv()
            xcf = pltpu.make_async_remote_copy(
                src_ref=o_ref.at[pl.ds(c0 + c * R, R), :],
                dst_ref=o_ref.at[pl.ds(c0 + c * R, R), :],
                send_sem=xc_s.at[XC.index(c)],
                recv_sem=xc_r.at[XC.index(c)],
                device_id=xnbr,
                device_id_type=pl.DeviceIdType.MESH,
            )
            xcf.start()
            return xcf

        acps = {}
        bfwd = {}
        cfwd = {}
        other_rdmas = []
        for c in range(NC):
            slot = c % 2
            if c + 1 < NC:
                pending[c + 1] = stage_in(c + 1)
            cp_p, cp_r = pending.pop(c)
            cp_p.wait()
            cp_r.wait()
            ypart_rdmas[c].wait_recv()
            ysum = p_st[slot] + prec_buf[c] + r_st[slot]
            ms = jnp.mean(ysum * ysum, axis=-1, keepdims=True)
            a_buf[c] = ysum * lax.rsqrt(ms + EPS) * g_ref[...]
            cp_a = pltpu.make_async_copy(
                a_buf.at[c],
                o_ref.at[pl.ds(a0 + c * R, R), :],
                acp.at[c])
            cp_a.start()
            acps[c] = cp_a
            xr = pltpu.make_async_remote_copy(
                src_ref=a_buf.at[c],
                dst_ref=xrecv_buf.at[c],
                send_sem=x_s.at[c],
                recv_sem=xrecv.at[c],
                device_id=xnbr,
                device_id_type=pl.DeviceIdType.MESH,
            )
            xr.start()
            other_rdmas.append(xr)
            if c in YA:
                yr = pltpu.make_async_remote_copy(
                    src_ref=a_buf.at[c],
                    dst_ref=o_ref.at[pl.ds(a0 + c * R, R), :],
                    send_sem=ya_s.at[YA.index(c)],
                    recv_sem=ya_r.at[YA.index(c)],
                    device_id=ynbr,
                    device_id_type=pl.DeviceIdType.MESH,
                )
                yr.start()
                other_rdmas.append(yr)
            else:
                zr = pltpu.make_async_remote_copy(
                    src_ref=a_buf.at[c],
                    dst_ref=o_ref.at[pl.ds(a0 + c * R, R), :],
                    send_sem=za_s.at[ZA.index(c)],
                    recv_sem=za_r.at[ZA.index(c)],
                    device_id=znbr,
                    device_id_type=pl.DeviceIdType.MESH,
                )
                zr.start()
                other_rdmas.append(zr)
            if c >= 1:
                bfwd[c - 1] = forward_b(c - 1)
            if c >= 2 and (c - 2) in XC:
                cfwd[c - 2] = forward_c(c - 2)
        bfwd[NC - 1] = forward_b(NC - 1)

        for c in range(NC):
            acps[c].wait()
            cp, zb = bfwd[c]
            cp.wait()
            if zb is not None:
                zb.wait_send()
            ypart_rdmas[c].wait_send()
        for c in XC:
            cfwd[c].wait_send()
        for rr in other_rdmas:
            rr.wait_send()
        src = a_buf.at[0]
        for c in YA:
            if c not in XC:
                ya_recv_desc(c).wait_recv()
        for c in ZA:
            pltpu.make_async_remote_copy(
                src_ref=src,
                dst_ref=o_ref.at[pl.ds(c0 + c * R, R), :],
                send_sem=ypart_s.at[c],
                recv_sem=za_r.at[ZA.index(c)],
                device_id=znbr,
                device_id_type=pl.DeviceIdType.MESH,
            ).wait_recv()
        for b in ZB:
            pltpu.make_async_remote_copy(
                src_ref=src,
                dst_ref=o_ref.at[pl.ds(d0 + b * R, R), :],
                send_sem=ypart_s.at[b],
                recv_sem=zb_r.at[ZB.index(b)],
                device_id=znbr,
                device_id_type=pl.DeviceIdType.MESH,
            ).wait_recv()
        for c in XC:
            pltpu.make_async_remote_copy(
                src_ref=src,
                dst_ref=o_ref.at[pl.ds(d0 + c * R, R), :],
                send_sem=ypart_s.at[c],
                recv_sem=xc_r.at[XC.index(c)],
                device_id=xnbr,
                device_id_type=pl.DeviceIdType.MESH,
            ).wait_recv()

    return pl.pallas_call(
        body,
        out_shape=jax.ShapeDtypeStruct((M, D), jnp.float32),
        in_specs=[
            pl.BlockSpec(memory_space=pl.ANY),
            pl.BlockSpec(memory_space=pl.ANY),
            pl.BlockSpec(memory_space=pltpu.VMEM),
        ],
        out_specs=pl.BlockSpec(memory_space=pl.ANY),
        scratch_shapes=[
            pltpu.VMEM((NC, R, D), jnp.float32),
            pltpu.VMEM((NC, R, D), jnp.float32),
            pltpu.VMEM((NC, R, D), jnp.float32),
            pltpu.VMEM((2, R, D), jnp.float32),
            pltpu.VMEM((2, R, D), jnp.float32),
            pltpu.SemaphoreType.DMA((NC,)),
            pltpu.SemaphoreType.DMA((NC,)),
            pltpu.SemaphoreType.DMA((NC,)),
            pltpu.SemaphoreType.DMA((NC,)),
            pltpu.SemaphoreType.DMA((len(YA),)),
            pltpu.SemaphoreType.DMA((len(YA),)),
            pltpu.SemaphoreType.DMA((len(ZA),)),
            pltpu.SemaphoreType.DMA((len(ZA),)),
            pltpu.SemaphoreType.DMA((len(ZB),)),
            pltpu.SemaphoreType.DMA((len(ZB),)),
            pltpu.SemaphoreType.DMA((len(XC),)),
            pltpu.SemaphoreType.DMA((len(XC),)),
            pltpu.SemaphoreType.DMA((2,)),
            pltpu.SemaphoreType.DMA((2,)),
            pltpu.SemaphoreType.DMA((NC,)),
            pltpu.SemaphoreType.DMA((NC,)),
        ],
        compiler_params=pltpu.CompilerParams(
            collective_id=0,
            vmem_limit_bytes=100 * 1024 * 1024,
        ),
    )(partial, resid, gamma2)


# device time: 348672 ns/iter; 1.4448x vs baseline; 1.4448x over previous
import jax
import jax.numpy as jnp
from jax import lax
from jax.experimental import pallas as pl
from jax.experimental.pallas import tpu as pltpu

M = 8192
D = 2048
EPS = 1e-6
Q = M // 4
NC = 8
R = Q // NC

YA = [0, 1, 2]
ZA = [3, 4, 5, 6, 7]
ZB = [0, 1, 2, 5, 6, 7]
XC = [3, 4]


def kernel(partial, resid, gamma):
    gamma2 = gamma.reshape(1, D)

    def body(p_ref, resid_ref, g_ref, o_ref,
             prec_buf, xrecv_buf, a_buf, p_st, r_st,
             ypart_s, yprec, x_s, xrecv,
             ya_s, ya_r, za_s, za_r, zb_s, zb_r, xc_s, xc_r,
             pin, rin, acp, bcp):
        x = lax.axis_index("x")
        y = lax.axis_index("y")
        z = lax.axis_index("z")
        xnbr = (1 - x, y, z)
        ynbr = (x, 1 - y, z)
        znbr = (x, y, 1 - z)
        h = jnp.bitwise_xor(y, z)
        a0 = (2 * x + h) * Q
        b0 = (2 * (1 - x) + h) * Q
        c0 = (2 * x + (1 - h)) * Q
        d0 = (2 * (1 - x) + (1 - h)) * Q

        def stage_in(c):
            slot = c % 2
            cp_p = pltpu.make_async_copy(
                p_ref.at[0, pl.ds(a0 + c * R, R), :],
                p_st.at[slot], pin.at[slot])
            cp_r = pltpu.make_async_copy(
                resid_ref.at[pl.ds(a0 + c * R, R), :],
                r_st.at[slot], rin.at[slot])
            cp_p.start()
            cp_r.start()
            return (cp_p, cp_r)

        pending = {0: stage_in(0)}

        bar = pltpu.get_barrier_semaphore()
        for nbr in (xnbr, ynbr, znbr):
            pl.semaphore_signal(bar, inc=1, device_id=nbr,
                                device_id_type=pl.DeviceIdType.MESH)
        pl.semaphore_wait(bar, 3)

        ypart_rdmas = []
        for c in range(NC):
            r = pltpu.make_async_remote_copy(
                src_ref=p_ref.at[0, pl.ds(c0 + c * R, R), :],
                dst_ref=prec_buf.at[c],
                send_sem=ypart_s.at[c],
                recv_sem=yprec.at[c],
                device_id=ynbr,
                device_id_type=pl.DeviceIdType.MESH,
            )
            r.start()
            ypart_rdmas.append(r)

        def xrecv_desc(b):
            return pltpu.make_async_remote_copy(
                src_ref=a_buf.at[b],
                dst_ref=xrecv_buf.at[b],
                send_sem=x_s.at[b],
                recv_sem=xrecv.at[b],
                device_id=xnbr,
                device_id_type=pl.DeviceIdType.MESH,
            )

        def forward_b(b):
            xrecv_desc(b).wait_recv()
            cp = pltpu.make_async_copy(
                xrecv_buf.at[b],
                o_ref.at[pl.ds(b0 + b * R, R), :],
                bcp.at[b])
            cp.start()
            zb = None
            if b in ZB:
                zb = pltpu.make_async_remote_copy(
                    src_ref=xrecv_buf.at[b],
                    dst_ref=o_ref.at[pl.ds(b0 + b * R, R), :],
                    send_sem=zb_s.at[ZB.index(b)],
                    recv_sem=zb_r.at[ZB.index(b)],
                    device_id=znbr,
                    device_id_type=pl.DeviceIdType.MESH,
                )
                zb.start()
            return (cp, zb)

        def ya_recv_desc(c):
            return pltpu.make_async_remote_copy(
                src_ref=a_buf.at[0],
                dst_ref=o_ref.at[pl.ds(c0 + c * R, R), :],
                send_sem=ypart_s.at[c],
                recv_sem=ya_r.at[YA.index(c)],
                device_id=ynbr,
                device_id_type=pl.DeviceIdType.MESH,
            )

        def za_recv_desc(c):
            return pltpu.make_async_remote_copy(
                src_ref=a_buf.at[0],
                dst_ref=o_ref.at[pl.ds(c0 + c * R, R), :],
                send_sem=ypart_s.at[c],
                recv_sem=za_r.at[ZA.index(c)],
                device_id=znbr,
                device_id_type=pl.DeviceIdType.MESH,
            )

        def forward_c(c):
            za_recv_desc(c).wait_recv()
            xcf = pltpu.make_async_remote_copy(
                src_ref=o_ref.at[pl.ds(c0 + c * R, R), :],
                dst_ref=o_ref.at[pl.ds(c0 + c * R, R), :],
                send_sem=xc_s.at[XC.index(c)],
                recv_sem=xc_r.at[XC.index(c)],
                device_id=xnbr,
                device_id_type=pl.DeviceIdType.MESH,
            )
            xcf.start()
            return xcf

        acps = {}
        bfwd = {}
        cfwd = {}
        other_rdmas = []
        for c in range(NC):
            slot = c % 2
            if c + 1 < NC:
                pending[c + 1] = stage_in(c + 1)
            cp_p, cp_r = pending.pop(c)
            cp_p.wait()
            cp_r.wait()
            ypart_rdmas[c].wait_recv()
            ysum = p_st[slot] + prec_buf[c] + r_st[slot]
            ms = jnp.mean(ysum * ysum, axis=-1, keepdims=True)
            a_buf[c] = ysum * lax.rsqrt(ms + EPS) * g_ref[...]
            cp_a = pltpu.make_async_copy(
                a_buf.at[c],
                o_ref.at[pl.ds(a0 + c * R, R), :],
                acp.at[c])
            cp_a.start()
            acps[c] = cp_a
            xr = pltpu.make_async_remote_copy(
                src_ref=a_buf.at[c],
                dst_ref=xrecv_buf.at[c],
                send_sem=x_s.at[c],
                recv_sem=xrecv.at[c],
                device_id=xnbr,
                device_id_type=pl.DeviceIdType.MESH,
            )
            xr.start()
            other_rdmas.append(xr)
            if c in YA:
                yr = pltpu.make_async_remote_copy(
                    src_ref=a_buf.at[c],
                    dst_ref=o_ref.at[pl.ds(a0 + c * R, R), :],
                    send_sem=ya_s.at[YA.index(c)],
                    recv_sem=ya_r.at[YA.index(c)],
                    device_id=ynbr,
                    device_id_type=pl.DeviceIdType.MESH,
                )
                yr.start()
                other_rdmas.append(yr)
            else:
                zr = pltpu.make_async_remote_copy(
                    src_ref=a_buf.at[c],
                    dst_ref=o_ref.at[pl.ds(a0 + c * R, R), :],
                    send_sem=za_s.at[ZA.index(c)],
                    recv_sem=za_r.at[ZA.index(c)],
                    device_id=znbr,
                    device_id_type=pl.DeviceIdType.MESH,
                )
                zr.start()
                other_rdmas.append(zr)
            if c >= 1:
                bfwd[c - 1] = forward_b(c - 1)
            if c >= 2 and (c - 2) in XC:
                cfwd[c - 2] = forward_c(c - 2)
        bfwd[NC - 1] = forward_b(NC - 1)

        for c in range(NC):
            acps[c].wait()
            cp, zb = bfwd[c]
            cp.wait()
            if zb is not None:
                zb.wait_send()
            ypart_rdmas[c].wait_send()
        for c in XC:
            cfwd[c].wait_send()
        for rr in other_rdmas:
            rr.wait_send()
        src = a_buf.at[0]
        for c in YA:
            ya_recv_desc(c).wait_recv()
        for c in ZA:
            if c not in XC:
                za_recv_desc(c).wait_recv()
        for b in ZB:
            pltpu.make_async_remote_copy(
                src_ref=src,
                dst_ref=o_ref.at[pl.ds(d0 + b * R, R), :],
                send_sem=ypart_s.at[b],
                recv_sem=zb_r.at[ZB.index(b)],
                device_id=znbr,
                device_id_type=pl.DeviceIdType.MESH,
            ).wait_recv()
        for c in XC:
            pltpu.make_async_remote_copy(
                src_ref=src,
                dst_ref=o_ref.at[pl.ds(d0 + c * R, R), :],
                send_sem=ypart_s.at[c],
                recv_sem=xc_r.at[XC.index(c)],
                device_id=xnbr,
                device_id_type=pl.DeviceIdType.MESH,
            ).wait_recv()

    return pl.pallas_call(
        body,
        out_shape=jax.ShapeDtypeStruct((M, D), jnp.float32),
        in_specs=[
            pl.BlockSpec(memory_space=pl.ANY),
            pl.BlockSpec(memory_space=pl.ANY),
            pl.BlockSpec(memory_space=pltpu.VMEM),
        ],
        out_specs=pl.BlockSpec(memory_space=pl.ANY),
        scratch_shapes=[
            pltpu.VMEM((NC, R, D), jnp.float32),
            pltpu.VMEM((NC, R, D), jnp.float32),
            pltpu.VMEM((NC, R, D), jnp.float32),
            pltpu.VMEM((2, R, D), jnp.float32),
            pltpu.VMEM((2, R, D), jnp.float32),
            pltpu.SemaphoreType.DMA((NC,)),
            pltpu.SemaphoreType.DMA((NC,)),
            pltpu.SemaphoreType.DMA((NC,)),
            pltpu.SemaphoreType.DMA((NC,)),
            pltpu.SemaphoreType.DMA((len(YA),)),
            pltpu.SemaphoreType.DMA((len(YA),)),
            pltpu.SemaphoreType.DMA((len(ZA),)),
            pltpu.SemaphoreType.DMA((len(ZA),)),
            pltpu.SemaphoreType.DMA((len(ZB),)),
            pltpu.SemaphoreType.DMA((len(ZB),)),
            pltpu.SemaphoreType.DMA((len(XC),)),
            pltpu.SemaphoreType.DMA((len(XC),)),
            pltpu.SemaphoreType.DMA((2,)),
            pltpu.SemaphoreType.DMA((2,)),
            pltpu.SemaphoreType.DMA((NC,)),
            pltpu.SemaphoreType.DMA((NC,)),
        ],
        compiler_params=pltpu.CompilerParams(
            collective_id=0,
            vmem_limit_bytes=100 * 1024 * 1024,
        ),
    )(partial, resid, gamma2)


# device time: 325793 ns/iter; 1.5463x vs baseline; 1.0702x over previous
import jax
import jax.numpy as jnp
from jax import lax
from jax.experimental import pallas as pl
from jax.experimental.pallas import tpu as pltpu

M = 8192
D = 2048
EPS = 1e-6
Q = M // 4
NC = 8
R = Q // NC

YA = [0, 1, 2]
ZA = [3, 4, 5, 6, 7]
ZB = [0, 1, 2, 6, 7]
XC = [3, 4, 5]


def kernel(partial, resid, gamma):
    gamma2 = gamma.reshape(1, D)

    def body(p_ref, resid_ref, g_ref, o_ref,
             prec_buf, xrecv_buf, a_buf, p_st, r_st,
             ypart_s, yprec, x_s, xrecv,
             ya_s, ya_r, za_s, za_r, zb_s, zb_r, xc_s, xc_r,
             pin, rin, acp, bcp):
        x = lax.axis_index("x")
        y = lax.axis_index("y")
        z = lax.axis_index("z")
        xnbr = (1 - x, y, z)
        ynbr = (x, 1 - y, z)
        znbr = (x, y, 1 - z)
        h = jnp.bitwise_xor(y, z)
        a0 = (2 * x + h) * Q
        b0 = (2 * (1 - x) + h) * Q
        c0 = (2 * x + (1 - h)) * Q
        d0 = (2 * (1 - x) + (1 - h)) * Q

        def stage_in(c):
            slot = c % 2
            cp_p = pltpu.make_async_copy(
                p_ref.at[0, pl.ds(a0 + c * R, R), :],
                p_st.at[slot], pin.at[slot])
            cp_r = pltpu.make_async_copy(
                resid_ref.at[pl.ds(a0 + c * R, R), :],
                r_st.at[slot], rin.at[slot])
            cp_p.start()
            cp_r.start()
            return (cp_p, cp_r)

        pending = {0: stage_in(0)}

        bar = pltpu.get_barrier_semaphore()
        for nbr in (xnbr, ynbr, znbr):
            pl.semaphore_signal(bar, inc=1, device_id=nbr,
                                device_id_type=pl.DeviceIdType.MESH)
        pl.semaphore_wait(bar, 3)

        ypart_rdmas = []
        for c in range(NC):
            r = pltpu.make_async_remote_copy(
                src_ref=p_ref.at[0, pl.ds(c0 + c * R, R), :],
                dst_ref=prec_buf.at[c],
                send_sem=ypart_s.at[c],
                recv_sem=yprec.at[c],
                device_id=ynbr,
                device_id_type=pl.DeviceIdType.MESH,
            )
            r.start()
            ypart_rdmas.append(r)

        def xrecv_desc(b):
            return pltpu.make_async_remote_copy(
                src_ref=a_buf.at[b],
                dst_ref=xrecv_buf.at[b],
                send_sem=x_s.at[b],
                recv_sem=xrecv.at[b],
                device_id=xnbr,
                device_id_type=pl.DeviceIdType.MESH,
            )

        def forward_b(b):
            xrecv_desc(b).wait_recv()
            cp = pltpu.make_async_copy(
                xrecv_buf.at[b],
                o_ref.at[pl.ds(b0 + b * R, R), :],
                bcp.at[b])
            cp.start()
            zb = None
            if b in ZB:
                zb = pltpu.make_async_remote_copy(
                    src_ref=xrecv_buf.at[b],
                    dst_ref=o_ref.at[pl.ds(b0 + b * R, R), :],
                    send_sem=zb_s.at[ZB.index(b)],
                    recv_sem=zb_r.at[ZB.index(b)],
                    device_id=znbr,
                    device_id_type=pl.DeviceIdType.MESH,
                )
                zb.start()
            return (cp, zb)

        def ya_recv_desc(c):
            return pltpu.make_async_remote_copy(
                src_ref=a_buf.at[0],
                dst_ref=o_ref.at[pl.ds(c0 + c * R, R), :],
                send_sem=ypart_s.at[c],
                recv_sem=ya_r.at[YA.index(c)],
                device_id=ynbr,
                device_id_type=pl.DeviceIdType.MESH,
            )

        def za_recv_desc(c):
            return pltpu.make_async_remote_copy(
                src_ref=a_buf.at[0],
                dst_ref=o_ref.at[pl.ds(c0 + c * R, R), :],
                send_sem=ypart_s.at[c],
                recv_sem=za_r.at[ZA.index(c)],
                device_id=znbr,
                device_id_type=pl.DeviceIdType.MESH,
            )

        def forward_c(c):
            za_recv_desc(c).wait_recv()
            xcf = pltpu.make_async_remote_copy(
                src_ref=o_ref.at[pl.ds(c0 + c * R, R), :],
                dst_ref=o_ref.at[pl.ds(c0 + c * R, R), :],
                send_sem=xc_s.at[XC.index(c)],
                recv_sem=xc_r.at[XC.index(c)],
                device_id=xnbr,
                device_id_type=pl.DeviceIdType.MESH,
            )
            xcf.start()
            return xcf

        acps = {}
        bfwd = {}
        cfwd = {}
        other_rdmas = []
        for c in range(NC):
            slot = c % 2
            if c + 1 < NC:
                pending[c + 1] = stage_in(c + 1)
            cp_p, cp_r = pending.pop(c)
            cp_p.wait()
            cp_r.wait()
            ypart_rdmas[c].wait_recv()
            ysum = p_st[slot] + prec_buf[c] + r_st[slot]
            ms = jnp.mean(ysum * ysum, axis=-1, keepdims=True)
            a_buf[c] = ysum * lax.rsqrt(ms + EPS) * g_ref[...]
            cp_a = pltpu.make_async_copy(
                a_buf.at[c],
                o_ref.at[pl.ds(a0 + c * R, R), :],
                acp.at[c])
            cp_a.start()
            acps[c] = cp_a
            xr = pltpu.make_async_remote_copy(
                src_ref=a_buf.at[c],
                dst_ref=xrecv_buf.at[c],
                send_sem=x_s.at[c],
                recv_sem=xrecv.at[c],
                device_id=xnbr,
                device_id_type=pl.DeviceIdType.MESH,
            )
            xr.start()
            other_rdmas.append(xr)
            if c in YA:
                yr = pltpu.make_async_remote_copy(
                    src_ref=a_buf.at[c],
                    dst_ref=o_ref.at[pl.ds(a0 + c * R, R), :],
                    send_sem=ya_s.at[YA.index(c)],
                    recv_sem=ya_r.at[YA.index(c)],
                    device_id=ynbr,
                    device_id_type=pl.DeviceIdType.MESH,
                )
                yr.start()
                other_rdmas.append(yr)
            else:
                zr = pltpu.make_async_remote_copy(
                    src_ref=a_buf.at[c],
                    dst_ref=o_ref.at[pl.ds(a0 + c * R, R), :],
                    send_sem=za_s.at[ZA.index(c)],
                    recv_sem=za_r.at[ZA.index(c)],
                    device_id=znbr,
                    device_id_type=pl.DeviceIdType.MESH,
                )
                zr.start()
                other_rdmas.append(zr)
            if c >= 1:
                bfwd[c - 1] = forward_b(c - 1)
            if c >= 2 and (c - 2) in XC:
                cfwd[c - 2] = forward_c(c - 2)
        bfwd[NC - 1] = forward_b(NC - 1)

        for c in range(NC):
            acps[c].wait()
            cp, zb = bfwd[c]
            cp.wait()
            if zb is not None:
                zb.wait_send()
            ypart_rdmas[c].wait_send()
        for c in XC:
            cfwd[c].wait_send()
        for rr in other_rdmas:
            rr.wait_send()
        src = a_buf.at[0]
        for c in YA:
            ya_recv_desc(c).wait_recv()
        for c in ZA:
            if c not in XC:
                za_recv_desc(c).wait_recv()
        for b in ZB:
            pltpu.make_async_remote_copy(
                src_ref=src,
                dst_ref=o_ref.at[pl.ds(d0 + b * R, R), :],
                send_sem=ypart_s.at[b],
                recv_sem=zb_r.at[ZB.index(b)],
                device_id=znbr,
                device_id_type=pl.DeviceIdType.MESH,
            ).wait_recv()
        for c in XC:
            pltpu.make_async_remote_copy(
                src_ref=src,
                dst_ref=o_ref.at[pl.ds(d0 + c * R, R), :],
                send_sem=ypart_s.at[c],
                recv_sem=xc_r.at[XC.index(c)],
                device_id=xnbr,
                device_id_type=pl.DeviceIdType.MESH,
            ).wait_recv()

    return pl.pallas_call(
        body,
        out_shape=jax.ShapeDtypeStruct((M, D), jnp.float32),
        in_specs=[
            pl.BlockSpec(memory_space=pl.ANY),
            pl.BlockSpec(memory_space=pl.ANY),
            pl.BlockSpec(memory_space=pltpu.VMEM),
        ],
        out_specs=pl.BlockSpec(memory_space=pl.ANY),
        scratch_shapes=[
            pltpu.VMEM((NC, R, D), jnp.float32),
            pltpu.VMEM((NC, R, D), jnp.float32),
            pltpu.VMEM((NC, R, D), jnp.float32),
            pltpu.VMEM((2, R, D), jnp.float32),
            pltpu.VMEM((2, R, D), jnp.float32),
            pltpu.SemaphoreType.DMA((NC,)),
            pltpu.SemaphoreType.DMA((NC,)),
            pltpu.SemaphoreType.DMA((NC,)),
            pltpu.SemaphoreType.DMA((NC,)),
            pltpu.SemaphoreType.DMA((len(YA),)),
            pltpu.SemaphoreType.DMA((len(YA),)),
            pltpu.SemaphoreType.DMA((len(ZA),)),
            pltpu.SemaphoreType.DMA((len(ZA),)),
            pltpu.SemaphoreType.DMA((len(ZB),)),
            pltpu.SemaphoreType.DMA((len(ZB),)),
            pltpu.SemaphoreType.DMA((len(XC),)),
            pltpu.SemaphoreType.DMA((len(XC),)),
            pltpu.SemaphoreType.DMA((2,)),
            pltpu.SemaphoreType.DMA((2,)),
            pltpu.SemaphoreType.DMA((NC,)),
            pltpu.SemaphoreType.DMA((NC,)),
        ],
        compiler_params=pltpu.CompilerParams(
            collective_id=0,
            vmem_limit_bytes=100 * 1024 * 1024,
        ),
    )(partial, resid, gamma2)
